# baseline (speedup 1.0000x reference)
import functools

import jax
import jax.numpy as jnp
from jax import lax
from jax.experimental import pallas as pl
from jax.experimental.pallas import tpu as pltpu
from jax.experimental.pallas import tpu_sc as plsc

B, T, D = 1, 2048, 1024
H = 1408
E = 8
K = 2
N = B * T
A = N * K
BA = 256
NB = A // BA + E
P = NB * BA
BT = 256
A2 = BT * K

NC, NS = 2, 16
NW = NC * NS
TPW = N // NW
CT = 32


def _silu(v):
    return v * jax.nn.sigmoid(v)


def _mm(a, b):
    return jax.lax.dot(a.astype(jnp.bfloat16), b.astype(jnp.bfloat16),
                       preferred_element_type=jnp.float32)


def _router_body(x_ref, wr_ref, wgs_ref, wus_ref, wds_ref,
                 ti0_ref, ti1_ref, rk0_ref, rk1_ref,
                 tp0_ref, tp1_ref, p0_ref, p1_ref, be_ref, rb_ref, nb_ref,
                 xb16_ref, sh_ref,
                 cnt_scr, wgs_b, wus_b, wds_b, ti0_s, ti1_s, rk0_s, rk1_s):
    t = pl.program_id(0)

    @pl.when(t == 0)
    def _prep():
        cnt_scr[...] = jnp.zeros((E, 1), jnp.float32)
        wgs_b[...] = wgs_ref[...].astype(jnp.bfloat16)
        wus_b[...] = wus_ref[...].astype(jnp.bfloat16)
        wds_b[...] = wds_ref[...].astype(jnp.bfloat16)

    xb = x_ref[...]
    xb16 = xb.astype(jnp.bfloat16)
    xb16_ref[...] = xb16
    shh = (_silu(jax.lax.dot(xb16, wgs_b[...],
                             preferred_element_type=jnp.float32))
           * jax.lax.dot(xb16, wus_b[...],
                         preferred_element_type=jnp.float32))
    sh_ref[...] = jax.lax.dot(shh.astype(jnp.bfloat16), wds_b[...],
                              preferred_element_type=jnp.float32)
    lt = jax.lax.dot_general(wr_ref[...], xb, (((0,), (1,)), ((), ())),
                             preferred_element_type=jnp.float32)
    mx = jnp.max(lt, axis=0, keepdims=True)
    ex = jnp.exp(lt - mx)
    probs = ex / jnp.sum(ex, axis=0, keepdims=True)
    ie = jax.lax.broadcasted_iota(jnp.int32, (E, BT), 0)
    m1 = jnp.max(probs, axis=0, keepdims=True)
    i1 = jnp.min(jnp.where(probs == m1, ie, E), axis=0, keepdims=True)
    p2 = jnp.where(ie == i1, -1.0, probs)
    m2 = jnp.max(p2, axis=0, keepdims=True)
    i2 = jnp.min(jnp.where(p2 == m2, ie, E), axis=0, keepdims=True)
    s = m1 + m2 + 1e-9
    tp0_ref[...] = m1 / s
    tp1_ref[...] = m2 / s
    ti0_ref[...] = i1
    ti1_ref[...] = i2

    @pl.when(t == 0)
    def _init():
        cnt_scr[...] = jnp.zeros((E, 1), jnp.float32)

    oh0 = (ie == i1).astype(jnp.float32)
    oh1 = (ie == i2).astype(jnp.float32)
    rr = jax.lax.broadcasted_iota(jnp.int32, (BT, BT), 0)
    cc = jax.lax.broadcasted_iota(jnp.int32, (BT, BT), 1)
    striu = (rr < cc).astype(jnp.float32)
    cnt = cnt_scr[...]
    tot0 = jnp.sum(oh0, axis=1, keepdims=True)
    r0 = (jnp.sum(jax.lax.dot(oh0, striu,
                              preferred_element_type=jnp.float32) * oh0,
                  axis=0, keepdims=True)
          + jnp.sum(cnt * oh0, axis=0, keepdims=True))
    r1 = (jnp.sum(jax.lax.dot(oh1, striu,
                              preferred_element_type=jnp.float32) * oh1,
                  axis=0, keepdims=True)
          + jnp.sum((cnt + tot0) * oh1, axis=0, keepdims=True))
    rk0i = r0.astype(jnp.int32)
    rk1i = r1.astype(jnp.int32)
    rk0_ref[...] = rk0i
    rk1_ref[...] = rk1i
    cnt_scr[...] = cnt + tot0 + jnp.sum(oh1, axis=1, keepdims=True)
    row = pl.ds(t * BT, BT)
    ti0_s[:, row] = i1
    ti1_s[:, row] = i2
    rk0_s[:, row] = rk0i
    rk1_s[:, row] = rk1i

    @pl.when(t == N // BT - 1)
    def _finalize():
        pci = (cnt_scr[...].astype(jnp.int32) + (BA - 1)) // BA
        t0 = ti0_s[...]
        t1 = ti1_s[...]
        acc0 = jnp.zeros((1, N), jnp.int32)
        acc1 = jnp.zeros((1, N), jnp.int32)
        bi = jax.lax.broadcasted_iota(jnp.int32, (1, NB), 1)
        be = jnp.zeros((1, NB), jnp.int32)
        bs = jnp.zeros((1, NB), jnp.int32)
        cum = jnp.zeros((1, 1), jnp.int32)
        for e in range(E):
            bstart_e = cum
            acc0 = acc0 + jnp.where(t0 == e, bstart_e, 0)
            acc1 = acc1 + jnp.where(t1 == e, bstart_e, 0)
            cum = cum + pci[e:e + 1, :]
            sel = bi >= cum
            be = be + jnp.where(sel & (be == e), 1, 0)
            bs = jnp.where(sel, cum, bs)
        p0_ref[...] = acc0 * BA + rk0_s[...]
        p1_ref[...] = acc1 * BA + rk1_s[...]
        be_ref[...] = jnp.minimum(be, E - 1)
        rb_ref[...] = (bi - bs) * BA
        nb_ref[...] = cum


def _router(flat, Wr, Wg_s, Wu_s, Wd_s):
    row_spec = pl.BlockSpec((1, BT), lambda t: (0, t))
    row_shape_i = jax.ShapeDtypeStruct((1, N), jnp.int32)
    row_shape_f = jax.ShapeDtypeStruct((1, N), jnp.float32)
    return pl.pallas_call(
        _router_body,
        grid=(N // BT,),
        in_specs=[
            pl.BlockSpec((BT, D), lambda t: (t, 0)),
            pl.BlockSpec((D, E), lambda t: (0, 0)),
            pl.BlockSpec((D, H), lambda t: (0, 0)),
            pl.BlockSpec((D, H), lambda t: (0, 0)),
            pl.BlockSpec((H, D), lambda t: (0, 0)),
        ],
        out_specs=[
            row_spec, row_spec, row_spec, row_spec, row_spec, row_spec,
            pl.BlockSpec((1, N), lambda t: (0, 0)),
            pl.BlockSpec((1, N), lambda t: (0, 0)),
            pl.BlockSpec((1, NB), lambda t: (0, 0)),
            pl.BlockSpec((1, NB), lambda t: (0, 0)),
            pl.BlockSpec((1, 1), lambda t: (0, 0)),
            pl.BlockSpec((BT, D), lambda t: (t, 0)),
            pl.BlockSpec((BT, D), lambda t: (t, 0)),
        ],
        out_shape=[
            row_shape_i, row_shape_i, row_shape_i, row_shape_i,
            row_shape_f, row_shape_f,
            jax.ShapeDtypeStruct((1, N), jnp.int32),
            jax.ShapeDtypeStruct((1, N), jnp.int32),
            jax.ShapeDtypeStruct((1, NB), jnp.int32),
            jax.ShapeDtypeStruct((1, NB), jnp.int32),
            jax.ShapeDtypeStruct((1, 1), jnp.int32),
            jax.ShapeDtypeStruct((N, D), jnp.bfloat16),
            jax.ShapeDtypeStruct((N, D), jnp.float32),
        ],
        scratch_shapes=[
            pltpu.VMEM((E, 1), jnp.float32),
            pltpu.VMEM((D, H), jnp.bfloat16),
            pltpu.VMEM((D, H), jnp.bfloat16),
            pltpu.VMEM((H, D), jnp.bfloat16),
            pltpu.VMEM((1, N), jnp.int32),
            pltpu.VMEM((1, N), jnp.int32),
            pltpu.VMEM((1, N), jnp.int32),
            pltpu.VMEM((1, N), jnp.int32),
        ],
        compiler_params=pltpu.CompilerParams(
            dimension_semantics=("arbitrary",),
        ),
    )(flat, Wr, Wg_s, Wu_s, Wd_s)


def _ffn_body(be_ref, rb_ref, nb_ref, ti0_ref, ti1_ref, rk0_ref, rk1_ref,
              tp0_ref, tp1_ref, xb_ref, wg_ref, wu_ref, wd_ref, ys_ref,
              wg_b, wu_b, wd_b):
    i = pl.program_id(0)

    @pl.when(i < nb_ref[0])
    def _():
        fresh = (i == 0) | (be_ref[i] != be_ref[jnp.maximum(i - 1, 0)])

        @pl.when(fresh)
        def _cast():
            wg_b[...] = wg_ref[0].astype(jnp.bfloat16)
            wu_b[...] = wu_ref[0].astype(jnp.bfloat16)
            wd_b[...] = wd_ref[0].astype(jnp.bfloat16)

        e = be_ref[i]
        rb = rb_ref[i]
        rows = jax.lax.broadcasted_iota(jnp.int32, (BA, N), 0) + rb
        c0 = (ti0_ref[...] == e) & (rk0_ref[...] == rows)
        c1 = (ti1_ref[...] == e) & (rk1_ref[...] == rows)
        oh = c0.astype(jnp.bfloat16) + c1.astype(jnp.bfloat16)
        xs = jax.lax.dot(oh, xb_ref[...],
                         preferred_element_type=jnp.float32)
        wgt = jnp.sum(jnp.where(c0, tp0_ref[...], 0.0)
                      + jnp.where(c1, tp1_ref[...], 0.0),
                      axis=1, keepdims=True)
        xsb = xs.astype(jnp.bfloat16)
        hg = jax.lax.dot(xsb, wg_b[...], preferred_element_type=jnp.float32)
        hu = jax.lax.dot(xsb, wu_b[...], preferred_element_type=jnp.float32)
        y = jax.lax.dot((_silu(hg) * hu).astype(jnp.bfloat16), wd_b[...],
                        preferred_element_type=jnp.float32)
        ys_ref[...] = y * wgt


def _grouped_ffn(blk_exp, rbase, nb_used, meta_rows, xb16, Wg, Wu, Wd):
    grid_spec = pltpu.PrefetchScalarGridSpec(
        num_scalar_prefetch=3,
        grid=(NB,),
        in_specs=(
            [pl.BlockSpec((1, N), lambda i, be, rb, nb: (0, 0))] * 6
            + [
                pl.BlockSpec((N, D), lambda i, be, rb, nb: (0, 0)),
                pl.BlockSpec((1, D, H), lambda i, be, rb, nb: (be[i], 0, 0)),
                pl.BlockSpec((1, D, H), lambda i, be, rb, nb: (be[i], 0, 0)),
                pl.BlockSpec((1, H, D), lambda i, be, rb, nb: (be[i], 0, 0)),
            ]
        ),
        out_specs=pl.BlockSpec((BA, D), lambda i, be, rb, nb: (i, 0)),
        scratch_shapes=[
            pltpu.VMEM((D, H), jnp.bfloat16),
            pltpu.VMEM((D, H), jnp.bfloat16),
            pltpu.VMEM((H, D), jnp.bfloat16),
        ],
    )
    return pl.pallas_call(
        _ffn_body,
        grid_spec=grid_spec,
        out_shape=jax.ShapeDtypeStruct((P, D), jnp.float32),
        compiler_params=pltpu.CompilerParams(
            dimension_semantics=("arbitrary",),
        ),
    )(blk_exp, rbase, nb_used, *meta_rows, xb16, Wg, Wu, Wd)


def _combine_body(p0_hbm, p1_hbm, sh_hbm, ys_hbm, out_hbm,
                  i0_v, i1_v, a_v, b_v, s_v, sem):
    wid = lax.axis_index("s") * NC + lax.axis_index("c")
    base = wid * TPW
    for c in range(TPW // CT):
        tb = base + c * CT
        pltpu.sync_copy(p0_hbm.at[pl.ds(tb, CT)], i0_v)
        pltpu.sync_copy(p1_hbm.at[pl.ds(tb, CT)], i1_v)
        cpa = pltpu.async_copy(ys_hbm.at[i0_v], a_v, sem)
        cpb = pltpu.async_copy(ys_hbm.at[i1_v], b_v, sem)
        pltpu.sync_copy(sh_hbm.at[pl.ds(tb, CT)], s_v)
        cpa.wait()
        cpb.wait()

        def _row(r, _):
            def _vec(j, _):
                sl = pl.ds(j * 16, 16)
                s_v[r, sl] = s_v[r, sl] + a_v[r, sl] + b_v[r, sl]
                return 0
            return lax.fori_loop(0, D // 16, _vec, 0, unroll=4)

        lax.fori_loop(0, CT, _row, 0)
        pltpu.sync_copy(s_v, out_hbm.at[pl.ds(tb, CT)])


def _combine(p0, p1, shared_out, ys):
    mesh = plsc.VectorSubcoreMesh(core_axis_name="c", subcore_axis_name="s")
    f = functools.partial(
        pl.kernel,
        mesh=mesh,
        out_type=jax.ShapeDtypeStruct((N, D), jnp.float32),
        scratch_types=[
            pltpu.VMEM((CT,), jnp.int32),
            pltpu.VMEM((CT,), jnp.int32),
            pltpu.VMEM((CT, D), jnp.float32),
            pltpu.VMEM((CT, D), jnp.float32),
            pltpu.VMEM((CT, D), jnp.float32),
            pltpu.SemaphoreType.DMA,
        ],
    )(_combine_body)
    return f(p0, p1, shared_out, ys)


def kernel(x, Wg_s, Wu_s, Wd_s, Wr, Wg, Wu, Wd):
    flat = x.reshape(N, D)
    (ti0, ti1, rk0, rk1, tp0, tp1, p0, p1, be, rb, nb, xb16,
     shared_out) = _router(flat, Wr, Wg_s, Wu_s, Wd_s)
    meta_rows = (ti0, ti1, rk0, rk1, tp0, tp1)
    ys = _grouped_ffn(be.reshape(NB), rb.reshape(NB), nb.reshape(1),
                      meta_rows, xb16, Wg, Wu, Wd)
    out = _combine(p0.reshape(N), p1.reshape(N), shared_out, ys)
    return out.reshape(x.shape)


# speedup vs baseline: 1.0320x; 1.0320x over previous
import functools

import jax
import jax.numpy as jnp
from jax import lax
from jax.experimental import pallas as pl
from jax.experimental.pallas import tpu as pltpu
from jax.experimental.pallas import tpu_sc as plsc

B, T, D = 1, 2048, 1024
H = 1408
E = 8
K = 2
N = B * T
A = N * K
BA = 256
NB = A // BA + E
P = NB * BA
BT = 256
A2 = BT * K

NC, NS = 2, 16
NW = NC * NS
TPW = N // NW
CT = 32


def _silu(v):
    return v * jax.nn.sigmoid(v)


def _mm(a, b):
    return jax.lax.dot(a.astype(jnp.bfloat16), b.astype(jnp.bfloat16),
                       preferred_element_type=jnp.float32)


def _router_body(x_ref, wr_ref, wgs_ref, wus_ref, wds_ref,
                 ti0_ref, ti1_ref, rk0_ref, rk1_ref,
                 tp0_ref, tp1_ref, p0_ref, p1_ref, be_ref, rb_ref, nb_ref,
                 xb16_ref, sh_ref,
                 cnt_scr, wgs_b, wus_b, wds_b, ti0_s, ti1_s, rk0_s, rk1_s):
    t = pl.program_id(0)

    @pl.when(t == 0)
    def _prep():
        cnt_scr[...] = jnp.zeros((E, 1), jnp.float32)
        wgs_b[...] = wgs_ref[...].astype(jnp.bfloat16)
        wus_b[...] = wus_ref[...].astype(jnp.bfloat16)
        wds_b[...] = wds_ref[...].astype(jnp.bfloat16)

    xb = x_ref[...]
    xb16 = xb.astype(jnp.bfloat16)
    xb16_ref[...] = xb16
    shh = (_silu(jax.lax.dot(xb16, wgs_b[...],
                             preferred_element_type=jnp.float32))
           * jax.lax.dot(xb16, wus_b[...],
                         preferred_element_type=jnp.float32))
    sh_ref[...] = jax.lax.dot(shh.astype(jnp.bfloat16), wds_b[...],
                              preferred_element_type=jnp.float32)
    lt = jax.lax.dot_general(wr_ref[...], xb, (((0,), (1,)), ((), ())),
                             preferred_element_type=jnp.float32)
    mx = jnp.max(lt, axis=0, keepdims=True)
    ex = jnp.exp(lt - mx)
    probs = ex / jnp.sum(ex, axis=0, keepdims=True)
    ie = jax.lax.broadcasted_iota(jnp.int32, (E, BT), 0)
    m1 = jnp.max(probs, axis=0, keepdims=True)
    i1 = jnp.min(jnp.where(probs == m1, ie, E), axis=0, keepdims=True)
    p2 = jnp.where(ie == i1, -1.0, probs)
    m2 = jnp.max(p2, axis=0, keepdims=True)
    i2 = jnp.min(jnp.where(p2 == m2, ie, E), axis=0, keepdims=True)
    s = m1 + m2 + 1e-9
    tp0_ref[...] = m1 / s
    tp1_ref[...] = m2 / s
    ti0_ref[...] = i1
    ti1_ref[...] = i2

    @pl.when(t == 0)
    def _init():
        cnt_scr[...] = jnp.zeros((E, 1), jnp.float32)

    oh0 = (ie == i1).astype(jnp.float32)
    oh1 = (ie == i2).astype(jnp.float32)
    rr = jax.lax.broadcasted_iota(jnp.int32, (BT, BT), 0)
    cc = jax.lax.broadcasted_iota(jnp.int32, (BT, BT), 1)
    striu = (rr < cc).astype(jnp.float32)
    cnt = cnt_scr[...]
    tot0 = jnp.sum(oh0, axis=1, keepdims=True)
    r0 = (jnp.sum(jax.lax.dot(oh0, striu,
                              preferred_element_type=jnp.float32) * oh0,
                  axis=0, keepdims=True)
          + jnp.sum(cnt * oh0, axis=0, keepdims=True))
    r1 = (jnp.sum(jax.lax.dot(oh1, striu,
                              preferred_element_type=jnp.float32) * oh1,
                  axis=0, keepdims=True)
          + jnp.sum((cnt + tot0) * oh1, axis=0, keepdims=True))
    rk0i = r0.astype(jnp.int32)
    rk1i = r1.astype(jnp.int32)
    rk0_ref[...] = rk0i
    rk1_ref[...] = rk1i
    cnt_scr[...] = cnt + tot0 + jnp.sum(oh1, axis=1, keepdims=True)
    row = pl.ds(t * BT, BT)
    ti0_s[:, row] = i1
    ti1_s[:, row] = i2
    rk0_s[:, row] = rk0i
    rk1_s[:, row] = rk1i

    @pl.when(t == N // BT - 1)
    def _finalize():
        pci = (cnt_scr[...].astype(jnp.int32) + (BA - 1)) // BA
        t0 = ti0_s[...]
        t1 = ti1_s[...]
        acc0 = jnp.zeros((1, N), jnp.int32)
        acc1 = jnp.zeros((1, N), jnp.int32)
        bi = jax.lax.broadcasted_iota(jnp.int32, (1, NB), 1)
        be = jnp.zeros((1, NB), jnp.int32)
        bs = jnp.zeros((1, NB), jnp.int32)
        cum = jnp.zeros((1, 1), jnp.int32)
        for e in range(E):
            bstart_e = cum
            acc0 = acc0 + jnp.where(t0 == e, bstart_e, 0)
            acc1 = acc1 + jnp.where(t1 == e, bstart_e, 0)
            cum = cum + pci[e:e + 1, :]
            sel = bi >= cum
            be = be + jnp.where(sel & (be == e), 1, 0)
            bs = jnp.where(sel, cum, bs)
        p0_ref[...] = acc0 * BA + rk0_s[...]
        p1_ref[...] = acc1 * BA + rk1_s[...]
        be_ref[...] = jnp.minimum(be, E - 1)
        rb_ref[...] = (bi - bs) * BA
        nb_ref[...] = cum


def _router(flat, Wr, Wg_s, Wu_s, Wd_s):
    row_spec = pl.BlockSpec((1, BT), lambda t: (0, t))
    row_shape_i = jax.ShapeDtypeStruct((1, N), jnp.int32)
    row_shape_f = jax.ShapeDtypeStruct((1, N), jnp.float32)
    return pl.pallas_call(
        _router_body,
        grid=(N // BT,),
        in_specs=[
            pl.BlockSpec((BT, D), lambda t: (t, 0)),
            pl.BlockSpec((D, E), lambda t: (0, 0)),
            pl.BlockSpec((D, H), lambda t: (0, 0)),
            pl.BlockSpec((D, H), lambda t: (0, 0)),
            pl.BlockSpec((H, D), lambda t: (0, 0)),
        ],
        out_specs=[
            row_spec, row_spec, row_spec, row_spec, row_spec, row_spec,
            pl.BlockSpec((1, N), lambda t: (0, 0)),
            pl.BlockSpec((1, N), lambda t: (0, 0)),
            pl.BlockSpec((1, NB), lambda t: (0, 0)),
            pl.BlockSpec((1, NB), lambda t: (0, 0)),
            pl.BlockSpec((1, 1), lambda t: (0, 0)),
            pl.BlockSpec((BT, D), lambda t: (t, 0)),
            pl.BlockSpec((BT, D), lambda t: (t, 0)),
        ],
        out_shape=[
            row_shape_i, row_shape_i, row_shape_i, row_shape_i,
            row_shape_f, row_shape_f,
            jax.ShapeDtypeStruct((1, N), jnp.int32),
            jax.ShapeDtypeStruct((1, N), jnp.int32),
            jax.ShapeDtypeStruct((1, NB), jnp.int32),
            jax.ShapeDtypeStruct((1, NB), jnp.int32),
            jax.ShapeDtypeStruct((1, 1), jnp.int32),
            jax.ShapeDtypeStruct((N, D), jnp.bfloat16),
            jax.ShapeDtypeStruct((N, D), jnp.float32),
        ],
        scratch_shapes=[
            pltpu.VMEM((E, 1), jnp.float32),
            pltpu.VMEM((D, H), jnp.bfloat16),
            pltpu.VMEM((D, H), jnp.bfloat16),
            pltpu.VMEM((H, D), jnp.bfloat16),
            pltpu.VMEM((1, N), jnp.int32),
            pltpu.VMEM((1, N), jnp.int32),
            pltpu.VMEM((1, N), jnp.int32),
            pltpu.VMEM((1, N), jnp.int32),
        ],
        compiler_params=pltpu.CompilerParams(
            dimension_semantics=("arbitrary",),
        ),
    )(flat, Wr, Wg_s, Wu_s, Wd_s)


def _ffn_body(be_ref, rb_ref, nb_ref, ti0_ref, ti1_ref, rk0_ref, rk1_ref,
              tp0_ref, tp1_ref, xb_ref, wg_ref, wu_ref, wd_ref, ys_ref):
    i = pl.program_id(0)

    @pl.when(i < nb_ref[0])
    def _():
        e = be_ref[i]
        rb = rb_ref[i]
        rows = jax.lax.broadcasted_iota(jnp.int32, (BA, N), 0) + rb
        c0 = (ti0_ref[...] == e) & (rk0_ref[...] == rows)
        c1 = (ti1_ref[...] == e) & (rk1_ref[...] == rows)
        oh = c0.astype(jnp.bfloat16) + c1.astype(jnp.bfloat16)
        xs = jax.lax.dot(oh, xb_ref[...],
                         preferred_element_type=jnp.float32)
        wgt = jnp.sum(jnp.where(c0, tp0_ref[...], 0.0)
                      + jnp.where(c1, tp1_ref[...], 0.0),
                      axis=1, keepdims=True)
        hg = _mm(xs, wg_ref[0])
        hu = _mm(xs, wu_ref[0])
        y = _mm(_silu(hg) * hu, wd_ref[0])
        ys_ref[...] = y * wgt


def _grouped_ffn(blk_exp, rbase, nb_used, meta_rows, xb16, Wg, Wu, Wd):
    grid_spec = pltpu.PrefetchScalarGridSpec(
        num_scalar_prefetch=3,
        grid=(NB,),
        in_specs=(
            [pl.BlockSpec((1, N), lambda i, be, rb, nb: (0, 0))] * 6
            + [
                pl.BlockSpec((N, D), lambda i, be, rb, nb: (0, 0)),
                pl.BlockSpec((1, D, H), lambda i, be, rb, nb: (be[i], 0, 0)),
                pl.BlockSpec((1, D, H), lambda i, be, rb, nb: (be[i], 0, 0)),
                pl.BlockSpec((1, H, D), lambda i, be, rb, nb: (be[i], 0, 0)),
            ]
        ),
        out_specs=pl.BlockSpec((BA, D), lambda i, be, rb, nb: (i, 0)),
    )
    return pl.pallas_call(
        _ffn_body,
        grid_spec=grid_spec,
        out_shape=jax.ShapeDtypeStruct((P, D), jnp.float32),
        compiler_params=pltpu.CompilerParams(
            dimension_semantics=("arbitrary",),
        ),
    )(blk_exp, rbase, nb_used, *meta_rows, xb16, Wg, Wu, Wd)


def _combine_body(p0_hbm, p1_hbm, sh_hbm, ys_hbm, out_hbm,
                  i0_v, i1_v, a_v, b_v, s_v, sem):
    wid = lax.axis_index("s") * NC + lax.axis_index("c")
    base = wid * TPW
    for c in range(TPW // CT):
        tb = base + c * CT
        pltpu.sync_copy(p0_hbm.at[pl.ds(tb, CT)], i0_v)
        pltpu.sync_copy(p1_hbm.at[pl.ds(tb, CT)], i1_v)
        cpa = pltpu.async_copy(ys_hbm.at[i0_v], a_v, sem)
        cpb = pltpu.async_copy(ys_hbm.at[i1_v], b_v, sem)
        pltpu.sync_copy(sh_hbm.at[pl.ds(tb, CT)], s_v)
        cpa.wait()
        cpb.wait()

        def _row(r, _):
            def _vec(j, _):
                sl = pl.ds(j * 16, 16)
                s_v[r, sl] = s_v[r, sl] + a_v[r, sl] + b_v[r, sl]
                return 0
            return lax.fori_loop(0, D // 16, _vec, 0, unroll=4)

        lax.fori_loop(0, CT, _row, 0)
        pltpu.sync_copy(s_v, out_hbm.at[pl.ds(tb, CT)])


def _combine(p0, p1, shared_out, ys):
    mesh = plsc.VectorSubcoreMesh(core_axis_name="c", subcore_axis_name="s")
    f = functools.partial(
        pl.kernel,
        mesh=mesh,
        out_type=jax.ShapeDtypeStruct((N, D), jnp.float32),
        scratch_types=[
            pltpu.VMEM((CT,), jnp.int32),
            pltpu.VMEM((CT,), jnp.int32),
            pltpu.VMEM((CT, D), jnp.float32),
            pltpu.VMEM((CT, D), jnp.float32),
            pltpu.VMEM((CT, D), jnp.float32),
            pltpu.SemaphoreType.DMA,
        ],
    )(_combine_body)
    return f(p0, p1, shared_out, ys)


def kernel(x, Wg_s, Wu_s, Wd_s, Wr, Wg, Wu, Wd):
    flat = x.reshape(N, D)
    (ti0, ti1, rk0, rk1, tp0, tp1, p0, p1, be, rb, nb, xb16,
     shared_out) = _router(flat, Wr, Wg_s, Wu_s, Wd_s)
    meta_rows = (ti0, ti1, rk0, rk1, tp0, tp1)
    ys = _grouped_ffn(be.reshape(NB), rb.reshape(NB), nb.reshape(1),
                      meta_rows, xb16, Wg, Wu, Wd)
    out = _combine(p0.reshape(N), p1.reshape(N), shared_out, ys)
    return out.reshape(x.shape)


# speedup vs baseline: 1.0357x; 1.0035x over previous
import functools

import jax
import jax.numpy as jnp
from jax import lax
from jax.experimental import pallas as pl
from jax.experimental.pallas import tpu as pltpu
from jax.experimental.pallas import tpu_sc as plsc

B, T, D = 1, 2048, 1024
H = 1408
E = 8
K = 2
N = B * T
A = N * K
BA = 256
NB = A // BA + E
P = NB * BA
BT = 256

NC, NS = 2, 16
NW = NC * NS
TPW = N // NW
CT = 32


def _silu(v):
    return v * jax.nn.sigmoid(v)


def _mm(a, b):
    return jax.lax.dot(a.astype(jnp.bfloat16), b.astype(jnp.bfloat16),
                       preferred_element_type=jnp.float32)


def _router_body(x_ref, wr_ref, wgs_ref, wus_ref, wds_ref,
                 ti0_ref, ti1_ref, rk0_ref, rk1_ref,
                 tp0_ref, tp1_ref, p0_ref, p1_ref, be_ref, rb_ref, nb_ref,
                 xb16_ref, sh_ref,
                 cnt_scr, wgs_b, wus_b, wds_b, ti0_s, ti1_s, rk0_s, rk1_s):
    t = pl.program_id(0)

    @pl.when(t == 0)
    def _prep():
        cnt_scr[...] = jnp.zeros((E, 1), jnp.float32)
        wgs_b[...] = wgs_ref[...].astype(jnp.bfloat16)
        wus_b[...] = wus_ref[...].astype(jnp.bfloat16)
        wds_b[...] = wds_ref[...].astype(jnp.bfloat16)

    xb = x_ref[...]
    xb16 = xb.astype(jnp.bfloat16)
    xb16_ref[...] = xb16
    shh = (_silu(jax.lax.dot(xb16, wgs_b[...],
                             preferred_element_type=jnp.float32))
           * jax.lax.dot(xb16, wus_b[...],
                         preferred_element_type=jnp.float32))
    sh_ref[...] = jax.lax.dot(shh.astype(jnp.bfloat16), wds_b[...],
                              preferred_element_type=jnp.float32)
    lt = jax.lax.dot_general(wr_ref[...], xb, (((0,), (1,)), ((), ())),
                             preferred_element_type=jnp.float32)
    mx = jnp.max(lt, axis=0, keepdims=True)
    ex = jnp.exp(lt - mx)
    probs = ex / jnp.sum(ex, axis=0, keepdims=True)
    ie = jax.lax.broadcasted_iota(jnp.int32, (E, BT), 0)
    m1 = jnp.max(probs, axis=0, keepdims=True)
    i1 = jnp.min(jnp.where(probs == m1, ie, E), axis=0, keepdims=True)
    p2 = jnp.where(ie == i1, -1.0, probs)
    m2 = jnp.max(p2, axis=0, keepdims=True)
    i2 = jnp.min(jnp.where(p2 == m2, ie, E), axis=0, keepdims=True)
    s = m1 + m2 + 1e-9
    tp0_ref[...] = m1 / s
    tp1_ref[...] = m2 / s
    ti0_ref[...] = i1
    ti1_ref[...] = i2

    @pl.when(t == 0)
    def _init():
        cnt_scr[...] = jnp.zeros((E, 1), jnp.float32)

    oh0 = (ie == i1).astype(jnp.float32)
    oh1 = (ie == i2).astype(jnp.float32)
    rr = jax.lax.broadcasted_iota(jnp.int32, (BT, BT), 0)
    cc = jax.lax.broadcasted_iota(jnp.int32, (BT, BT), 1)
    striu = (rr < cc).astype(jnp.float32)
    cnt = cnt_scr[...]
    tot0 = jnp.sum(oh0, axis=1, keepdims=True)
    r0 = (jnp.sum(jax.lax.dot(oh0, striu,
                              preferred_element_type=jnp.float32) * oh0,
                  axis=0, keepdims=True)
          + jnp.sum(cnt * oh0, axis=0, keepdims=True))
    r1 = (jnp.sum(jax.lax.dot(oh1, striu,
                              preferred_element_type=jnp.float32) * oh1,
                  axis=0, keepdims=True)
          + jnp.sum((cnt + tot0) * oh1, axis=0, keepdims=True))
    rk0i = r0.astype(jnp.int32)
    rk1i = r1.astype(jnp.int32)
    rk0_ref[...] = rk0i
    rk1_ref[...] = rk1i
    cnt_scr[...] = cnt + tot0 + jnp.sum(oh1, axis=1, keepdims=True)
    row = pl.ds(t * BT, BT)
    ti0_s[:, row] = i1
    ti1_s[:, row] = i2
    rk0_s[:, row] = rk0i
    rk1_s[:, row] = rk1i

    @pl.when(t == N // BT - 1)
    def _finalize():
        pci = (cnt_scr[...].astype(jnp.int32) + (BA - 1)) // BA
        t0 = ti0_s[...]
        t1 = ti1_s[...]
        acc0 = jnp.zeros((1, N), jnp.int32)
        acc1 = jnp.zeros((1, N), jnp.int32)
        bi = jax.lax.broadcasted_iota(jnp.int32, (1, NB), 1)
        be = jnp.zeros((1, NB), jnp.int32)
        bs = jnp.zeros((1, NB), jnp.int32)
        cum = jnp.zeros((1, 1), jnp.int32)
        for e in range(E):
            bstart_e = cum
            acc0 = acc0 + jnp.where(t0 == e, bstart_e, 0)
            acc1 = acc1 + jnp.where(t1 == e, bstart_e, 0)
            cum = cum + pci[e:e + 1, :]
            sel = bi >= cum
            be = be + jnp.where(sel & (be == e), 1, 0)
            bs = jnp.where(sel, cum, bs)
        p0_ref[...] = acc0 * BA + rk0_s[...]
        p1_ref[...] = acc1 * BA + rk1_s[...]
        be_ref[...] = jnp.minimum(be, E - 1)
        rb_ref[...] = (bi - bs) * BA
        nb_ref[...] = cum


def _router(flat, Wr, Wg_s, Wu_s, Wd_s):
    row_spec = pl.BlockSpec((1, BT), lambda t: (0, t))
    row_shape_i = jax.ShapeDtypeStruct((1, N), jnp.int32)
    row_shape_f = jax.ShapeDtypeStruct((1, N), jnp.float32)
    return pl.pallas_call(
        _router_body,
        grid=(N // BT,),
        in_specs=[
            pl.BlockSpec((BT, D), lambda t: (t, 0)),
            pl.BlockSpec((D, E), lambda t: (0, 0)),
            pl.BlockSpec((D, H), lambda t: (0, 0)),
            pl.BlockSpec((D, H), lambda t: (0, 0)),
            pl.BlockSpec((H, D), lambda t: (0, 0)),
        ],
        out_specs=[
            row_spec, row_spec, row_spec, row_spec, row_spec, row_spec,
            pl.BlockSpec((1, N), lambda t: (0, 0)),
            pl.BlockSpec((1, N), lambda t: (0, 0)),
            pl.BlockSpec((1, NB), lambda t: (0, 0)),
            pl.BlockSpec((1, NB), lambda t: (0, 0)),
            pl.BlockSpec((1, 1), lambda t: (0, 0)),
            pl.BlockSpec((BT, D), lambda t: (t, 0)),
            pl.BlockSpec((BT, D), lambda t: (t, 0)),
        ],
        out_shape=[
            row_shape_i, row_shape_i, row_shape_i, row_shape_i,
            row_shape_f, row_shape_f,
            jax.ShapeDtypeStruct((1, N), jnp.int32),
            jax.ShapeDtypeStruct((1, N), jnp.int32),
            jax.ShapeDtypeStruct((1, NB), jnp.int32),
            jax.ShapeDtypeStruct((1, NB), jnp.int32),
            jax.ShapeDtypeStruct((1, 1), jnp.int32),
            jax.ShapeDtypeStruct((N, D), jnp.bfloat16),
            jax.ShapeDtypeStruct((N, D), jnp.float32),
        ],
        scratch_shapes=[
            pltpu.VMEM((E, 1), jnp.float32),
            pltpu.VMEM((D, H), jnp.bfloat16),
            pltpu.VMEM((D, H), jnp.bfloat16),
            pltpu.VMEM((H, D), jnp.bfloat16),
            pltpu.VMEM((1, N), jnp.int32),
            pltpu.VMEM((1, N), jnp.int32),
            pltpu.VMEM((1, N), jnp.int32),
            pltpu.VMEM((1, N), jnp.int32),
        ],
        compiler_params=pltpu.CompilerParams(
            dimension_semantics=("arbitrary",),
        ),
    )(flat, Wr, Wg_s, Wu_s, Wd_s)


def _ffn_body(be_ref, rb_ref, nb_ref, ti0_ref, ti1_ref, rk0_ref, rk1_ref,
              tp0_ref, tp1_ref, xb_ref, wg_ref, wu_ref, wd_ref, ys_ref):
    i = pl.program_id(0)

    @pl.when(i < nb_ref[0])
    def _():
        e = be_ref[i]
        rb = rb_ref[i]
        rows = jax.lax.broadcasted_iota(jnp.int32, (BA, N), 0) + rb
        c0 = (ti0_ref[...] == e) & (rk0_ref[...] == rows)
        c1 = (ti1_ref[...] == e) & (rk1_ref[...] == rows)
        oh = c0.astype(jnp.bfloat16) + c1.astype(jnp.bfloat16)
        xs = jax.lax.dot(oh, xb_ref[...],
                         preferred_element_type=jnp.float32)
        wgt = jnp.sum(jnp.where(c0, tp0_ref[...], 0.0)
                      + jnp.where(c1, tp1_ref[...], 0.0),
                      axis=1, keepdims=True)
        hg = _mm(xs, wg_ref[0])
        hu = _mm(xs, wu_ref[0])
        y = _mm(_silu(hg) * hu, wd_ref[0])
        ys_ref[...] = y * wgt


def _grouped_ffn(blk_exp, rbase, nb_used, meta_rows, xb16, Wg, Wu, Wd):
    grid_spec = pltpu.PrefetchScalarGridSpec(
        num_scalar_prefetch=3,
        grid=(NB,),
        in_specs=(
            [pl.BlockSpec((1, N), lambda i, be, rb, nb: (0, 0))] * 6
            + [
                pl.BlockSpec((N, D), lambda i, be, rb, nb: (0, 0)),
                pl.BlockSpec((1, D, H), lambda i, be, rb, nb: (be[i], 0, 0)),
                pl.BlockSpec((1, D, H), lambda i, be, rb, nb: (be[i], 0, 0)),
                pl.BlockSpec((1, H, D), lambda i, be, rb, nb: (be[i], 0, 0)),
            ]
        ),
        out_specs=pl.BlockSpec((BA, D), lambda i, be, rb, nb: (i, 0)),
    )
    return pl.pallas_call(
        _ffn_body,
        grid_spec=grid_spec,
        out_shape=jax.ShapeDtypeStruct((P, D), jnp.float32),
        compiler_params=pltpu.CompilerParams(
            dimension_semantics=("arbitrary",),
        ),
    )(blk_exp, rbase, nb_used, *meta_rows, xb16, Wg, Wu, Wd)


def _combine_body(p0_hbm, p1_hbm, sh_hbm, ys_hbm, out_hbm,
                  i0_v, i1_v, a_v, b_v, s_v, sem):
    wid = lax.axis_index("s") * NC + lax.axis_index("c")
    base = wid * TPW
    for c in range(TPW // CT):
        tb = base + c * CT
        pltpu.sync_copy(p0_hbm.at[pl.ds(tb, CT)], i0_v)
        pltpu.sync_copy(p1_hbm.at[pl.ds(tb, CT)], i1_v)
        cpa = pltpu.async_copy(ys_hbm.at[i0_v], a_v, sem)
        cpb = pltpu.async_copy(ys_hbm.at[i1_v], b_v, sem)
        pltpu.sync_copy(sh_hbm.at[pl.ds(tb, CT)], s_v)
        cpa.wait()
        cpb.wait()

        def _row(r, _):
            def _vec(j, _):
                sl = pl.ds(j * 16, 16)
                s_v[r, sl] = s_v[r, sl] + a_v[r, sl] + b_v[r, sl]
                return 0
            return lax.fori_loop(0, D // 16, _vec, 0, unroll=4)

        lax.fori_loop(0, CT, _row, 0)
        pltpu.sync_copy(s_v, out_hbm.at[pl.ds(tb, CT)])


def _combine(p0, p1, shared_out, ys):
    mesh = plsc.VectorSubcoreMesh(core_axis_name="c", subcore_axis_name="s")
    f = functools.partial(
        pl.kernel,
        mesh=mesh,
        out_type=jax.ShapeDtypeStruct((N, D), jnp.float32),
        scratch_types=[
            pltpu.VMEM((CT,), jnp.int32),
            pltpu.VMEM((CT,), jnp.int32),
            pltpu.VMEM((CT, D), jnp.float32),
            pltpu.VMEM((CT, D), jnp.float32),
            pltpu.VMEM((CT, D), jnp.float32),
            pltpu.SemaphoreType.DMA,
        ],
    )(_combine_body)
    return f(p0, p1, shared_out, ys)


def kernel(x, Wg_s, Wu_s, Wd_s, Wr, Wg, Wu, Wd):
    flat = x.reshape(N, D)
    (ti0, ti1, rk0, rk1, tp0, tp1, p0, p1, be, rb, nb, xb16,
     shared_out) = _router(flat, Wr, Wg_s, Wu_s, Wd_s)
    meta_rows = (ti0, ti1, rk0, rk1, tp0, tp1)
    ys = _grouped_ffn(be.reshape(NB), rb.reshape(NB), nb.reshape(1),
                      meta_rows, xb16, Wg, Wu, Wd)
    out = _combine(p0.reshape(N), p1.reshape(N), shared_out, ys)
    return out.reshape(x.shape)


# speedup vs baseline: 1.0377x; 1.0019x over previous
import functools

import jax
import jax.numpy as jnp
from jax import lax
from jax.experimental import pallas as pl
from jax.experimental.pallas import tpu as pltpu
from jax.experimental.pallas import tpu_sc as plsc

B, T, D = 1, 2048, 1024
H = 1408
E = 8
K = 2
N = B * T
A = N * K
BA = 256
NB = A // BA + E
P = NB * BA
BT = 256

NC, NS = 2, 16
NW = NC * NS
TPW = N // NW
CT = 32


def _silu(v):
    return v * jax.nn.sigmoid(v)


def _mm(a, b):
    return jax.lax.dot(a.astype(jnp.bfloat16), b.astype(jnp.bfloat16),
                       preferred_element_type=jnp.float32)


def _router_body(x_ref, wr_ref, wgs_ref, wus_ref, wds_ref,
                 ti0_ref, ti1_ref, rk0_ref, rk1_ref,
                 tp0_ref, tp1_ref, p0_ref, p1_ref, be_ref, rb_ref, nb_ref,
                 xb16_ref, sh_ref,
                 cnt_scr, wgs_b, wus_b, wds_b, ti0_s, ti1_s, rk0_s, rk1_s):
    t = pl.program_id(0)

    @pl.when(t == 0)
    def _prep():
        cnt_scr[...] = jnp.zeros((E, 1), jnp.float32)
        wgs_b[...] = wgs_ref[...].astype(jnp.bfloat16)
        wus_b[...] = wus_ref[...].astype(jnp.bfloat16)
        wds_b[...] = wds_ref[...].astype(jnp.bfloat16)

    xb = x_ref[...]
    xb16 = xb.astype(jnp.bfloat16)
    xb16_ref[...] = xb16
    shh = (_silu(jax.lax.dot(xb16, wgs_b[...],
                             preferred_element_type=jnp.float32))
           * jax.lax.dot(xb16, wus_b[...],
                         preferred_element_type=jnp.float32))
    sh_ref[...] = jax.lax.dot(shh.astype(jnp.bfloat16), wds_b[...],
                              preferred_element_type=jnp.float32)
    lt = jax.lax.dot_general(wr_ref[...], xb, (((0,), (1,)), ((), ())),
                             preferred_element_type=jnp.float32)
    mx = jnp.max(lt, axis=0, keepdims=True)
    ex = jnp.exp(lt - mx)
    probs = ex / jnp.sum(ex, axis=0, keepdims=True)
    ie = jax.lax.broadcasted_iota(jnp.int32, (E, BT), 0)
    m1 = jnp.max(probs, axis=0, keepdims=True)
    i1 = jnp.min(jnp.where(probs == m1, ie, E), axis=0, keepdims=True)
    p2 = jnp.where(ie == i1, -1.0, probs)
    m2 = jnp.max(p2, axis=0, keepdims=True)
    i2 = jnp.min(jnp.where(p2 == m2, ie, E), axis=0, keepdims=True)
    s = m1 + m2 + 1e-9
    tp0_ref[...] = m1 / s
    tp1_ref[...] = m2 / s
    ti0_ref[...] = i1
    ti1_ref[...] = i2

    oh0 = (ie == i1).astype(jnp.float32)
    oh1 = (ie == i2).astype(jnp.float32)
    rr = jax.lax.broadcasted_iota(jnp.int32, (BT, BT), 0)
    cc = jax.lax.broadcasted_iota(jnp.int32, (BT, BT), 1)
    striu = (rr < cc).astype(jnp.float32)
    cnt = cnt_scr[...]
    tot0 = jnp.sum(oh0, axis=1, keepdims=True)
    r0 = (jnp.sum(jax.lax.dot(oh0, striu,
                              preferred_element_type=jnp.float32) * oh0,
                  axis=0, keepdims=True)
          + jnp.sum(cnt * oh0, axis=0, keepdims=True))
    r1 = (jnp.sum(jax.lax.dot(oh1, striu,
                              preferred_element_type=jnp.float32) * oh1,
                  axis=0, keepdims=True)
          + jnp.sum((cnt + tot0) * oh1, axis=0, keepdims=True))
    rk0i = r0.astype(jnp.int32)
    rk1i = r1.astype(jnp.int32)
    rk0_ref[...] = rk0i
    rk1_ref[...] = rk1i
    cnt_scr[...] = cnt + tot0 + jnp.sum(oh1, axis=1, keepdims=True)
    row = pl.ds(t * BT, BT)
    ti0_s[:, row] = i1
    ti1_s[:, row] = i2
    rk0_s[:, row] = rk0i
    rk1_s[:, row] = rk1i

    @pl.when(t == N // BT - 1)
    def _finalize():
        pci = (cnt_scr[...].astype(jnp.int32) + (BA - 1)) // BA
        t0 = ti0_s[...]
        t1 = ti1_s[...]
        acc0 = jnp.zeros((1, N), jnp.int32)
        acc1 = jnp.zeros((1, N), jnp.int32)
        bi = jax.lax.broadcasted_iota(jnp.int32, (1, NB), 1)
        be = jnp.zeros((1, NB), jnp.int32)
        bs = jnp.zeros((1, NB), jnp.int32)
        cum = jnp.zeros((1, 1), jnp.int32)
        for e in range(E):
            bstart_e = cum
            acc0 = acc0 + jnp.where(t0 == e, bstart_e, 0)
            acc1 = acc1 + jnp.where(t1 == e, bstart_e, 0)
            cum = cum + pci[e:e + 1, :]
            sel = bi >= cum
            be = be + jnp.where(sel & (be == e), 1, 0)
            bs = jnp.where(sel, cum, bs)
        p0_ref[...] = acc0 * BA + rk0_s[...]
        p1_ref[...] = acc1 * BA + rk1_s[...]
        be_ref[...] = jnp.minimum(be, E - 1)
        rb_ref[...] = (bi - bs) * BA
        nb_ref[...] = cum


def _router(flat, Wr, Wg_s, Wu_s, Wd_s):
    row_spec = pl.BlockSpec((1, BT), lambda t: (0, t))
    row_shape_i = jax.ShapeDtypeStruct((1, N), jnp.int32)
    row_shape_f = jax.ShapeDtypeStruct((1, N), jnp.float32)
    return pl.pallas_call(
        _router_body,
        grid=(N // BT,),
        in_specs=[
            pl.BlockSpec((BT, D), lambda t: (t, 0)),
            pl.BlockSpec((D, E), lambda t: (0, 0)),
            pl.BlockSpec((D, H), lambda t: (0, 0)),
            pl.BlockSpec((D, H), lambda t: (0, 0)),
            pl.BlockSpec((H, D), lambda t: (0, 0)),
        ],
        out_specs=[
            row_spec, row_spec, row_spec, row_spec, row_spec, row_spec,
            pl.BlockSpec((1, N), lambda t: (0, 0)),
            pl.BlockSpec((1, N), lambda t: (0, 0)),
            pl.BlockSpec((1, NB), lambda t: (0, 0)),
            pl.BlockSpec((1, NB), lambda t: (0, 0)),
            pl.BlockSpec((1, 1), lambda t: (0, 0)),
            pl.BlockSpec((BT, D), lambda t: (t, 0)),
            pl.BlockSpec((BT, D), lambda t: (t, 0)),
        ],
        out_shape=[
            row_shape_i, row_shape_i, row_shape_i, row_shape_i,
            row_shape_f, row_shape_f,
            jax.ShapeDtypeStruct((1, N), jnp.int32),
            jax.ShapeDtypeStruct((1, N), jnp.int32),
            jax.ShapeDtypeStruct((1, NB), jnp.int32),
            jax.ShapeDtypeStruct((1, NB), jnp.int32),
            jax.ShapeDtypeStruct((1, 1), jnp.int32),
            jax.ShapeDtypeStruct((N, D), jnp.bfloat16),
            jax.ShapeDtypeStruct((N, D), jnp.float32),
        ],
        scratch_shapes=[
            pltpu.VMEM((E, 1), jnp.float32),
            pltpu.VMEM((D, H), jnp.bfloat16),
            pltpu.VMEM((D, H), jnp.bfloat16),
            pltpu.VMEM((H, D), jnp.bfloat16),
            pltpu.VMEM((1, N), jnp.int32),
            pltpu.VMEM((1, N), jnp.int32),
            pltpu.VMEM((1, N), jnp.int32),
            pltpu.VMEM((1, N), jnp.int32),
        ],
        compiler_params=pltpu.CompilerParams(
            dimension_semantics=("arbitrary",),
        ),
    )(flat, Wr, Wg_s, Wu_s, Wd_s)


def _ffn_body(be_ref, rb_ref, nb_ref, ti0_ref, ti1_ref, rk0_ref, rk1_ref,
              tp0_ref, tp1_ref, xb_ref, wg_ref, wu_ref, wd_ref, ys_ref):
    i = pl.program_id(0)

    @pl.when(i < nb_ref[0])
    def _():
        e = be_ref[i]
        rb = rb_ref[i]
        rows = jax.lax.broadcasted_iota(jnp.int32, (BA, N), 0) + rb
        c0 = (ti0_ref[...] == e) & (rk0_ref[...] == rows)
        c1 = (ti1_ref[...] == e) & (rk1_ref[...] == rows)
        oh = c0.astype(jnp.bfloat16) + c1.astype(jnp.bfloat16)
        xs = jax.lax.dot(oh, xb_ref[...],
                         preferred_element_type=jnp.float32)
        wgt = jnp.sum(jnp.where(c0, tp0_ref[...], 0.0)
                      + jnp.where(c1, tp1_ref[...], 0.0),
                      axis=1, keepdims=True)
        hg = _mm(xs, wg_ref[0])
        hu = _mm(xs, wu_ref[0])
        y = _mm(_silu(hg) * hu, wd_ref[0])
        ys_ref[...] = y * wgt


def _grouped_ffn(blk_exp, rbase, nb_used, meta_rows, xb16, Wg, Wu, Wd):
    grid_spec = pltpu.PrefetchScalarGridSpec(
        num_scalar_prefetch=3,
        grid=(NB,),
        in_specs=(
            [pl.BlockSpec((1, N), lambda i, be, rb, nb: (0, 0))] * 6
            + [
                pl.BlockSpec((N, D), lambda i, be, rb, nb: (0, 0)),
                pl.BlockSpec((1, D, H), lambda i, be, rb, nb: (be[i], 0, 0)),
                pl.BlockSpec((1, D, H), lambda i, be, rb, nb: (be[i], 0, 0)),
                pl.BlockSpec((1, H, D), lambda i, be, rb, nb: (be[i], 0, 0)),
            ]
        ),
        out_specs=pl.BlockSpec((BA, D), lambda i, be, rb, nb: (i, 0)),
    )
    return pl.pallas_call(
        _ffn_body,
        grid_spec=grid_spec,
        out_shape=jax.ShapeDtypeStruct((P, D), jnp.float32),
        compiler_params=pltpu.CompilerParams(
            dimension_semantics=("arbitrary",),
        ),
    )(blk_exp, rbase, nb_used, *meta_rows, xb16, Wg, Wu, Wd)


def _combine_body(p0_hbm, p1_hbm, sh_hbm, ys_hbm, out_hbm,
                  i0_v, i1_v, a_v, b_v, s_v, sem):
    wid = lax.axis_index("s") * NC + lax.axis_index("c")
    base = wid * TPW
    for c in range(TPW // CT):
        tb = base + c * CT
        pltpu.sync_copy(p0_hbm.at[pl.ds(tb, CT)], i0_v)
        pltpu.sync_copy(p1_hbm.at[pl.ds(tb, CT)], i1_v)
        cpa = pltpu.async_copy(ys_hbm.at[i0_v], a_v, sem)
        cpb = pltpu.async_copy(ys_hbm.at[i1_v], b_v, sem)
        pltpu.sync_copy(sh_hbm.at[pl.ds(tb, CT)], s_v)
        cpa.wait()
        cpb.wait()

        def _row(r, _):
            def _vec(j, _):
                sl = pl.ds(j * 16, 16)
                s_v[r, sl] = s_v[r, sl] + a_v[r, sl] + b_v[r, sl]
                return 0
            return lax.fori_loop(0, D // 16, _vec, 0, unroll=4)

        lax.fori_loop(0, CT, _row, 0)
        pltpu.sync_copy(s_v, out_hbm.at[pl.ds(tb, CT)])


def _combine(p0, p1, shared_out, ys):
    mesh = plsc.VectorSubcoreMesh(core_axis_name="c", subcore_axis_name="s")
    f = functools.partial(
        pl.kernel,
        mesh=mesh,
        out_type=jax.ShapeDtypeStruct((N, D), jnp.float32),
        scratch_types=[
            pltpu.VMEM((CT,), jnp.int32),
            pltpu.VMEM((CT,), jnp.int32),
            pltpu.VMEM((CT, D), jnp.float32),
            pltpu.VMEM((CT, D), jnp.float32),
            pltpu.VMEM((CT, D), jnp.float32),
            pltpu.SemaphoreType.DMA,
        ],
    )(_combine_body)
    return f(p0, p1, shared_out, ys)


def kernel(x, Wg_s, Wu_s, Wd_s, Wr, Wg, Wu, Wd):
    flat = x.reshape(N, D)
    (ti0, ti1, rk0, rk1, tp0, tp1, p0, p1, be, rb, nb, xb16,
     shared_out) = _router(flat, Wr, Wg_s, Wu_s, Wd_s)
    meta_rows = (ti0, ti1, rk0, rk1, tp0, tp1)
    ys = _grouped_ffn(be.reshape(NB), rb.reshape(NB), nb.reshape(1),
                      meta_rows, xb16, Wg, Wu, Wd)
    out = _combine(p0.reshape(N), p1.reshape(N), shared_out, ys)
    return out.reshape(x.shape)
